# baseline (speedup 1.0000x reference)
import functools

import jax
import jax.numpy as jnp
from jax import lax
from jax.experimental import pallas as pl
from jax.experimental.pallas import tpu as pltpu
from jax.experimental.pallas import tpu_sc as plsc

N = 10000
E = 320000
D = 128

NC = 2
NS = 16
NW = NC * NS
EPW = E // NW
K = 80
NCHUNK = EPW // K
NPAIR = (NCHUNK - 1) // 2
RPT = 640
CPY = 80
NCPY = RPT // CPY
NCPY_LAST = (N - 15 * RPT) // CPY


def _sc_agg_body(h_hbm, src_hbm, dst_hbm, out_hbm,
                 acc_sh, src_full, dstA, dstB, rowsA, rowsB,
                 isemA, isemB, gsemA, gsemB):
    c = lax.axis_index("c")
    s = lax.axis_index("s")
    wid = c * NS + s
    base = wid * EPW

    pltpu.async_copy(src_hbm.at[pl.ds(base, EPW)], src_full, isemA)
    pltpu.async_copy(dst_hbm.at[pl.ds(base, K)], dstA, isemA)
    pltpu.async_copy(dst_hbm.at[pl.ds(base + K, K)], dstB, isemB)

    buf0 = rowsA.at[pl.ds(0, CPY), :]
    buf1 = rowsB.at[pl.ds(0, CPY), :]

    def init_from_h(ncpy):
        def hload(j, b, sem):
            pltpu.async_copy(h_hbm.at[pl.ds(s * RPT + j * CPY, CPY), :], b, sem)

        def hwait(b, sem):
            pltpu.make_async_copy(h_hbm.at[pl.ds(0, CPY), :], b, sem).wait()

        hload(0, buf0, gsemA)
        for j in range(ncpy):
            b, sem = (buf0, gsemA) if j % 2 == 0 else (buf1, gsemB)
            nb, nsem = (buf1, gsemB) if j % 2 == 0 else (buf0, gsemA)
            hwait(b, sem)
            if j + 1 < ncpy:
                hload(j + 1, nb, nsem)
            pltpu.sync_copy(b, acc_sh.at[pl.ds(s * RPT + j * CPY, CPY), :])

    def init_zero(ncpy):
        def zrow(r, carry):
            for cc in range(D // 16):
                rowsA[r, pl.ds(cc * 16, 16)] = jnp.zeros((16,), jnp.float32)
            return carry
        lax.fori_loop(0, CPY, zrow, 0)
        for j in range(ncpy):
            r0 = s * RPT + j * CPY
            pltpu.async_copy(buf0, acc_sh.at[pl.ds(r0, CPY), :], gsemA)
        for j in range(ncpy):
            pltpu.make_async_copy(buf0, acc_sh.at[pl.ds(0, CPY), :], gsemA).wait()

    is_last = s == NS - 1

    @pl.when(jnp.logical_and(c == 0, jnp.logical_not(is_last)))
    def _():
        init_from_h(NCPY)

    @pl.when(jnp.logical_and(c == 0, is_last))
    def _():
        init_from_h(NCPY_LAST)

    @pl.when(jnp.logical_and(c != 0, jnp.logical_not(is_last)))
    def _():
        init_zero(NCPY)

    @pl.when(jnp.logical_and(c != 0, is_last))
    def _():
        init_zero(NCPY_LAST)

    plsc.subcore_barrier()

    def istart(dv, isem, j):
        pltpu.async_copy(dst_hbm.at[pl.ds(base + j * K, K)], dv, isem)

    def iwait(dv, isem):
        pltpu.make_async_copy(dst_hbm.at[pl.ds(0, K)], dv, isem).wait()

    def gstart(j, rows, gsem):
        pltpu.async_copy(h_hbm.at[src_full.at[pl.ds(j * K, K)]], rows, gsem)

    def gwait(rows, gsem):
        pltpu.make_async_copy(h_hbm.at[pl.ds(0, K), :], rows, gsem).wait()

    def scat(rows, dv):
        pltpu.sync_copy(rows, acc_sh.at[dv], add=True)

    pltpu.make_async_copy(dst_hbm.at[pl.ds(0, EPW)], src_full, isemA).wait()
    iwait(dstA, isemA)
    gstart(0, rowsA, gsemA)

    def pair(i, carry):
        j2 = i * 2
        gwait(rowsA, gsemA)
        iwait(dstB, isemB)
        gstart(j2 + 1, rowsB, gsemB)
        scat(rowsA, dstA)
        istart(dstA, isemA, jnp.minimum(j2 + 2, NCHUNK - 1))
        gwait(rowsB, gsemB)
        iwait(dstA, isemA)
        gstart(jnp.minimum(j2 + 2, NCHUNK - 1), rowsA, gsemA)
        scat(rowsB, dstB)
        istart(dstB, isemB, jnp.minimum(j2 + 3, NCHUNK - 1))
        return carry

    lax.fori_loop(0, NPAIR, pair, 0)

    gwait(rowsA, gsemA)
    scat(rowsA, dstA)
    iwait(dstB, isemB)

    plsc.subcore_barrier()

    def copy_out(ncpy):
        def aread(j, b, sem):
            pltpu.async_copy(acc_sh.at[pl.ds(s * RPT + j * CPY, CPY), :], b, sem)

        def await_(b, sem):
            pltpu.make_async_copy(acc_sh.at[pl.ds(0, CPY), :], b, sem).wait()

        aread(0, buf0, gsemA)
        for j in range(ncpy):
            b, sem = (buf0, gsemA) if j % 2 == 0 else (buf1, gsemB)
            nb, nsem = (buf1, gsemB) if j % 2 == 0 else (buf0, gsemA)
            await_(b, sem)
            if j + 1 < ncpy:
                aread(j + 1, nb, nsem)
            pltpu.sync_copy(b, out_hbm.at[c, pl.ds(s * RPT + j * CPY, CPY), :])

    @pl.when(jnp.logical_not(is_last))
    def _():
        copy_out(NCPY)

    @pl.when(is_last)
    def _():
        copy_out(NCPY_LAST)


_sc_agg = pl.kernel(
    _sc_agg_body,
    out_type=jax.ShapeDtypeStruct((NC, N, D), jnp.float32),
    mesh=plsc.VectorSubcoreMesh(core_axis_name="c", subcore_axis_name="s",
                                num_cores=NC, num_subcores=NS),
    scratch_types=[
        pltpu.VMEM_SHARED((N, D), jnp.float32),
        pltpu.VMEM((EPW,), jnp.int32),
        pltpu.VMEM((K,), jnp.int32),
        pltpu.VMEM((K,), jnp.int32),
        pltpu.VMEM((K, D), jnp.float32),
        pltpu.VMEM((K, D), jnp.float32),
        pltpu.SemaphoreType.DMA,
        pltpu.SemaphoreType.DMA,
        pltpu.SemaphoreType.DMA,
        pltpu.SemaphoreType.DMA,
    ],
)

BN = 1000


def _mlp_body(relu_out, p_ref, w1_ref, b1_ref, w2_ref, b2_ref, o_ref):
    z = p_ref[0] + p_ref[1]
    t = jnp.maximum(
        jnp.dot(z, w1_ref[...], preferred_element_type=jnp.float32)
        + b1_ref[...], 0.0)
    o = jnp.dot(t, w2_ref[...], preferred_element_type=jnp.float32) + b2_ref[...]
    if relu_out:
        o = jnp.maximum(o, 0.0)
    o_ref[...] = o


def _mlp(p, w1, b1, w2, b2, relu_out):
    return pl.pallas_call(
        functools.partial(_mlp_body, relu_out),
        grid=(N // BN,),
        in_specs=[
            pl.BlockSpec((NC, BN, D), lambda i: (0, i, 0)),
            pl.BlockSpec((D, D), lambda i: (0, 0)),
            pl.BlockSpec((1, D), lambda i: (0, 0)),
            pl.BlockSpec((D, D), lambda i: (0, 0)),
            pl.BlockSpec((1, D), lambda i: (0, 0)),
        ],
        out_specs=pl.BlockSpec((BN, D), lambda i: (i, 0)),
        out_shape=jax.ShapeDtypeStruct((N, D), jnp.float32),
    )(p, w1, b1.reshape(1, D), w2, b2.reshape(1, D))


def kernel(x, edge_index, W1_0, b1_0, W2_0, b2_0, W1_1, b1_1, W2_1, b2_1):
    src = edge_index[0]
    dst = edge_index[1]
    p = _sc_agg(x, src, dst)
    h = _mlp(p, W1_0, b1_0, W2_0, b2_0, relu_out=True)
    q = _sc_agg(h, src, dst)
    out = _mlp(q, W1_1, b1_1, W2_1, b2_1, relu_out=False)
    return out


# speedup vs baseline: 1.0005x; 1.0005x over previous
import functools

import jax
import jax.numpy as jnp
from jax import lax
from jax.experimental import pallas as pl
from jax.experimental.pallas import tpu as pltpu
from jax.experimental.pallas import tpu_sc as plsc

N = 10000
E = 320000
D = 128

NC = 2
NS = 16
NW = NC * NS
EPW = E // NW
K = 80
NCHUNK = EPW // K
NPAIR = (NCHUNK - 1) // 2
RPT = 640
CPY = 80
NCPY = RPT // CPY
NCPY_LAST = (N - 15 * RPT) // CPY


def _sc_agg_body(h_hbm, src_hbm, dst_hbm, out_hbm,
                 acc_sh, src_full, dst0, dst1, dst2, dst3, rowsA, rowsB,
                 isem0, isem1, isem2, isem3, gsemA, gsemB, ssemA, ssemB):
    c = lax.axis_index("c")
    s = lax.axis_index("s")
    wid = c * NS + s
    base = wid * EPW

    dstV = [dst0, dst1, dst2, dst3]
    isemV = [isem0, isem1, isem2, isem3]
    rowsV = [rowsA, rowsB]
    gsemV = [gsemA, gsemB]
    ssemV = [ssemA, ssemB]

    pltpu.async_copy(src_hbm.at[pl.ds(base, EPW)], src_full, isem2)
    pltpu.async_copy(dst_hbm.at[pl.ds(base, K)], dst0, isem0)
    pltpu.async_copy(dst_hbm.at[pl.ds(base + K, K)], dst1, isem1)

    buf0 = rowsA.at[pl.ds(0, CPY), :]
    buf1 = rowsB.at[pl.ds(0, CPY), :]

    def init_from_h(ncpy):
        def hload(j, b, sem):
            pltpu.async_copy(h_hbm.at[pl.ds(s * RPT + j * CPY, CPY), :], b, sem)

        def hwait(b, sem):
            pltpu.make_async_copy(h_hbm.at[pl.ds(0, CPY), :], b, sem).wait()

        hload(0, buf0, gsemA)
        for j in range(ncpy):
            b, sem = (buf0, gsemA) if j % 2 == 0 else (buf1, gsemB)
            nb, nsem = (buf1, gsemB) if j % 2 == 0 else (buf0, gsemA)
            hwait(b, sem)
            if j + 1 < ncpy:
                hload(j + 1, nb, nsem)
            pltpu.sync_copy(b, acc_sh.at[pl.ds(s * RPT + j * CPY, CPY), :])

    def init_zero(ncpy):
        def zrow(r, carry):
            for cc in range(D // 16):
                rowsA[r, pl.ds(cc * 16, 16)] = jnp.zeros((16,), jnp.float32)
            return carry
        lax.fori_loop(0, CPY, zrow, 0)
        for j in range(ncpy):
            r0 = s * RPT + j * CPY
            pltpu.async_copy(buf0, acc_sh.at[pl.ds(r0, CPY), :], gsemA)
        for j in range(ncpy):
            pltpu.make_async_copy(buf0, acc_sh.at[pl.ds(0, CPY), :], gsemA).wait()

    is_last = s == NS - 1

    @pl.when(jnp.logical_and(c == 0, jnp.logical_not(is_last)))
    def _():
        init_from_h(NCPY)

    @pl.when(jnp.logical_and(c == 0, is_last))
    def _():
        init_from_h(NCPY_LAST)

    @pl.when(jnp.logical_and(c != 0, jnp.logical_not(is_last)))
    def _():
        init_zero(NCPY)

    @pl.when(jnp.logical_and(c != 0, is_last))
    def _():
        init_zero(NCPY_LAST)

    plsc.subcore_barrier()

    def istart(p4, j):
        pltpu.async_copy(dst_hbm.at[pl.ds(base + j * K, K)], dstV[p4], isemV[p4])

    def iwait(p4):
        pltpu.make_async_copy(dst_hbm.at[pl.ds(0, K)], dstV[p4],
                              isemV[p4]).wait()

    def gstart(j, p2):
        pltpu.async_copy(h_hbm.at[src_full.at[pl.ds(j * K, K)]], rowsV[p2],
                         gsemV[p2])

    def gwait(p2):
        pltpu.make_async_copy(h_hbm.at[pl.ds(0, K), :], rowsV[p2],
                              gsemV[p2]).wait()

    def sstart(p2, p4):
        pltpu.async_copy(rowsV[p2], acc_sh.at[dstV[p4]], ssemV[p2], add=True)

    def swait(p2):
        pltpu.make_async_copy(rowsV[p2], acc_sh.at[dstV[0]],
                              ssemV[p2]).wait()

    def phase(j, p2, p4, do_swait=True, do_next=True, inext=None):
        gwait(p2)
        iwait(p4)
        sstart(p2, p4)
        if do_swait:
            swait(1 - p2)
        if do_next:
            gstart(j + 1, 1 - p2)
            istart((p4 + 2) % 4, j + 2 if inext is None else inext)
        return j

    pltpu.make_async_copy(dst_hbm.at[pl.ds(0, EPW)], src_full, isem2).wait()
    gstart(0, 0)
    phase(0, 0, 0, do_swait=False)
    phase(1, 1, 1)

    def quad(i, carry):
        j = 2 + i * 4
        phase(j, 0, 2)
        phase(j + 1, 1, 3)
        phase(j + 2, 0, 0)
        phase(j + 3, 1, 1)
        return carry

    lax.fori_loop(0, (NCHUNK - 5) // 4, quad, 0)

    phase(NCHUNK - 3, 0, 2)
    phase(NCHUNK - 2, 1, 3, inext=NCHUNK - 1)
    phase(NCHUNK - 1, 0, 0, do_next=False)
    swait(0)
    iwait(1)

    plsc.subcore_barrier()

    def copy_out(ncpy):
        def aread(j, b, sem):
            pltpu.async_copy(acc_sh.at[pl.ds(s * RPT + j * CPY, CPY), :], b, sem)

        def await_(b, sem):
            pltpu.make_async_copy(acc_sh.at[pl.ds(0, CPY), :], b, sem).wait()

        aread(0, buf0, gsemA)
        for j in range(ncpy):
            b, sem = (buf0, gsemA) if j % 2 == 0 else (buf1, gsemB)
            nb, nsem = (buf1, gsemB) if j % 2 == 0 else (buf0, gsemA)
            await_(b, sem)
            if j + 1 < ncpy:
                aread(j + 1, nb, nsem)
            pltpu.sync_copy(b, out_hbm.at[c, pl.ds(s * RPT + j * CPY, CPY), :])

    @pl.when(jnp.logical_not(is_last))
    def _():
        copy_out(NCPY)

    @pl.when(is_last)
    def _():
        copy_out(NCPY_LAST)


_sc_agg = pl.kernel(
    _sc_agg_body,
    out_type=jax.ShapeDtypeStruct((NC, N, D), jnp.float32),
    mesh=plsc.VectorSubcoreMesh(core_axis_name="c", subcore_axis_name="s",
                                num_cores=NC, num_subcores=NS),
    scratch_types=[
        pltpu.VMEM_SHARED((N, D), jnp.float32),
        pltpu.VMEM((EPW,), jnp.int32),
        pltpu.VMEM((K,), jnp.int32),
        pltpu.VMEM((K,), jnp.int32),
        pltpu.VMEM((K,), jnp.int32),
        pltpu.VMEM((K,), jnp.int32),
        pltpu.VMEM((K, D), jnp.float32),
        pltpu.VMEM((K, D), jnp.float32),
        pltpu.SemaphoreType.DMA,
        pltpu.SemaphoreType.DMA,
        pltpu.SemaphoreType.DMA,
        pltpu.SemaphoreType.DMA,
        pltpu.SemaphoreType.DMA,
        pltpu.SemaphoreType.DMA,
        pltpu.SemaphoreType.DMA,
        pltpu.SemaphoreType.DMA,
    ],
)

BN = 1000


def _mlp_body(relu_out, p_ref, w1_ref, b1_ref, w2_ref, b2_ref, o_ref):
    z = p_ref[0] + p_ref[1]
    t = jnp.maximum(
        jnp.dot(z, w1_ref[...], preferred_element_type=jnp.float32)
        + b1_ref[...], 0.0)
    o = jnp.dot(t, w2_ref[...], preferred_element_type=jnp.float32) + b2_ref[...]
    if relu_out:
        o = jnp.maximum(o, 0.0)
    o_ref[...] = o


def _mlp(p, w1, b1, w2, b2, relu_out):
    return pl.pallas_call(
        functools.partial(_mlp_body, relu_out),
        grid=(N // BN,),
        in_specs=[
            pl.BlockSpec((NC, BN, D), lambda i: (0, i, 0)),
            pl.BlockSpec((D, D), lambda i: (0, 0)),
            pl.BlockSpec((1, D), lambda i: (0, 0)),
            pl.BlockSpec((D, D), lambda i: (0, 0)),
            pl.BlockSpec((1, D), lambda i: (0, 0)),
        ],
        out_specs=pl.BlockSpec((BN, D), lambda i: (i, 0)),
        out_shape=jax.ShapeDtypeStruct((N, D), jnp.float32),
    )(p, w1, b1.reshape(1, D), w2, b2.reshape(1, D))


def kernel(x, edge_index, W1_0, b1_0, W2_0, b2_0, W1_1, b1_1, W2_1, b2_1):
    src = edge_index[0]
    dst = edge_index[1]
    p = _sc_agg(x, src, dst)
    h = _mlp(p, W1_0, b1_0, W2_0, b2_0, relu_out=True)
    q = _sc_agg(h, src, dst)
    out = _mlp(q, W1_1, b1_1, W2_1, b2_1, relu_out=False)
    return out
